# baseline (speedup 1.0000x reference)
import functools

import jax
import jax.numpy as jnp
from jax import lax
from jax.experimental import pallas as pl
from jax.experimental.pallas import tpu as pltpu
from jax.experimental.pallas import tpu_sc as plsc


def _gather_call(seq, bt, d, nw, nc, idx4, wt2):
    g = bt // nw
    nbuf = 4
    half = 2
    mesh = plsc.VectorSubcoreMesh(core_axis_name="c", subcore_axis_name="s")
    scratch = [
        pltpu.VMEM((seq, g), jnp.int32),
        pltpu.VMEM((nbuf, g), jnp.int32),
        pltpu.VMEM((nbuf, g, 2 * d), jnp.float32),
        pltpu.VMEM((half, g, d), jnp.float32),
    ] + [pltpu.SemaphoreType.DMA] * (nbuf + half)

    @functools.partial(
        pl.kernel,
        mesh=mesh,
        out_type=jax.ShapeDtypeStruct((bt, seq, d), jnp.float32),
        compiler_params=pltpu.CompilerParams(use_tc_tiling_on_sc=True),
        scratch_types=scratch,
    )
    def k(idx_hbm, tab_hbm, out_hbm, idx_v, p_v, g_v, o_v, *sems):
        gsem = sems[:nbuf]
        osem = sems[nbuf:]
        wid = lax.axis_index("s") * nc + lax.axis_index("c")
        pltpu.sync_copy(idx_hbm.at[wid], idx_v)
        row0 = wid * g

        def fire_gather(si, slot):
            for t in range(g // 16):
                vv = idx_v[si, pl.ds(16 * t, 16)]
                p_v[slot, pl.ds(16 * t, 16)] = vv >> 1
            pltpu.make_async_copy(
                tab_hbm.at[p_v.at[slot]], g_v.at[slot], gsem[slot]
            ).start()

        def gather_wait(slot):
            pltpu.make_async_copy(
                tab_hbm.at[p_v.at[slot]], g_v.at[slot], gsem[slot]
            ).wait()

        def out_desc(si, oslot):
            return pltpu.make_async_copy(
                o_v.at[oslot],
                out_hbm.at[pl.ds(row0, g), si, :],
                osem[oslot],
            )

        def compact(si, slot, oslot):
            gref = g_v.at[slot]
            oref = o_v.at[oslot]

            def body(t, carry):
                vv = idx_v[si, pl.ds(16 * t, 16)]
                hb = (vv & 1) * d
                for i2 in range(16):
                    i = 16 * t + i2
                    hh = hb[i2]
                    for u in range(d // 16):
                        oref[i, pl.ds(16 * u, 16)] = gref[i, pl.ds(hh + 16 * u, 16)]
                return carry

            lax.fori_loop(0, g // 16, body, 0)

        for si in range(nbuf):
            fire_gather(si, si)

        def blk(bi, carry):
            for bsl in range(nbuf):
                si = bi * nbuf + bsl
                oslot = bsl % half
                gather_wait(bsl)

                @pl.when(si >= half)
                def _():
                    out_desc(lax.max(si - half, 0), oslot).wait()

                compact(si, bsl, oslot)
                out_desc(si, oslot).start()

                @pl.when(si + nbuf < seq)
                def _():
                    fire_gather(lax.min(si + nbuf, seq - 1), bsl)

            return carry

        lax.fori_loop(0, seq // nbuf, blk, 0)
        out_desc(seq - 2, 0).wait()
        out_desc(seq - 1, 1).wait()

    return k(idx4, wt2)


def kernel(token_ids, weight):
    bt, seq = token_ids.shape
    v, d = weight.shape
    info = plsc.get_sparse_core_info()
    nc, ns = info.num_cores, info.num_subcores
    nw = nc * ns
    g = bt // nw
    wt2 = weight.reshape(v // 2, 2 * d)
    idx4 = token_ids.reshape(nw, g, seq).transpose(0, 2, 1)
    out = _gather_call(seq, bt, d, nw, nc, idx4, wt2)
    return out
